# baseline (speedup 1.0000x reference)
import jax
import jax.numpy as jnp
from jax import lax
from jax.experimental import pallas as pl
from jax.experimental.pallas import tpu as pltpu
from jax.experimental.pallas import tpu_sc as plsc

_NCLS = 31
_BG = 30
_BUDGET = 80000.0
_PLANE = 512 * 512
_PX = 4 * _PLANE
_NW = 32
_PPW = _PX // _NW
_CHUNK = 1024
_NCH = _PPW // _CHUNK
_LANES = 16
_ROWS = 128


def _argmax_sc(inp_hbm, preds_hbm, buf0, buf1, pb0, pb1, si0, si1, so0, so1):
    cid = lax.axis_index("c")
    sid = lax.axis_index("s")
    wid = sid * 2 + cid
    base = wid * _PPW
    b = base // _PLANE
    off = base - b * _PLANE
    bufs = (buf0, buf1)
    pbs = (pb0, pb1)
    sis = (si0, si1)
    sos = (so0, so1)

    def in_copy(ci, k):
        return pltpu.make_async_copy(
            inp_hbm.at[b, :, pl.ds(off + ci * _CHUNK, _CHUNK)], bufs[k], sis[k])

    def out_copy(ci, k):
        return pltpu.make_async_copy(
            pbs[k], preds_hbm.at[pl.ds(base + ci * _CHUNK, _CHUNK)], sos[k])

    in_copy(0, 0).start()
    in_copy(1, 1).start()

    def pair(i2, carry):
        for k in range(2):
            ci = i2 * 2 + k
            in_copy(ci, k).wait()

            @pl.when(i2 >= 1)
            def _wait_out():
                out_copy(ci - 2, k).wait()

            buf = bufs[k]
            pb = pbs[k]

            def grp(j, c2):
                s = pl.ds(j * _LANES, _LANES)
                m = buf[0, s]
                idx = jnp.zeros((_LANES,), jnp.float32)
                for c in range(1, _NCLS):
                    v = buf[c, s]
                    gt = v > m
                    m = jnp.where(gt, v, m)
                    idx = jnp.where(gt, jnp.float32(c), idx)
                pb[s] = idx
                return c2

            lax.fori_loop(0, _CHUNK // _LANES, grp, 0)
            out_copy(ci, k).start()

            @pl.when(ci + 2 < _NCH)
            def _next_in():
                in_copy(ci + 2, k).start()

        return carry

    lax.fori_loop(0, _NCH // 2, pair, 0)
    out_copy(_NCH - 2, 0).wait()
    out_copy(_NCH - 1, 1).wait()


def _gate_tc(p_ref, t_ref, out_ref, acc_ref, cnt_ref):
    step = pl.program_id(0)
    nsteps = pl.num_programs(0)

    @pl.when(step == 0)
    def _init():
        acc_ref[0] = 0.0
        acc_ref[1] = 0.0
        cnt_ref[0] = 0

    p = p_ref[...]
    t = t_ref[...]
    tf = t.astype(jnp.float32)
    bg = t == _BG
    bgf = bg.astype(jnp.float32)

    i_nb = jnp.sum(jnp.where(bg, 0.0, p * tf))
    t_nb = jnp.sum(jnp.where(bg, 0.0, p + tf))

    rows, cols = bgf.shape
    jj = lax.broadcasted_iota(jnp.int32, (cols, cols), 0)
    kk = lax.broadcasted_iota(jnp.int32, (cols, cols), 1)
    tri_inc = (jj <= kk).astype(jnp.float32)
    cs_in = jnp.dot(bgf, tri_inc, preferred_element_type=jnp.float32)
    row_tot = cs_in[:, cols - 1:cols]
    ii = lax.broadcasted_iota(jnp.int32, (rows, rows), 0)
    ll = lax.broadcasted_iota(jnp.int32, (rows, rows), 1)
    tri_lo = (ll < ii).astype(jnp.float32)
    r_pref = jnp.dot(tri_lo, row_tot, preferred_element_type=jnp.float32)
    rank_ex = r_pref + (cs_in - bgf)

    offset = cnt_ref[0].astype(jnp.float32)
    include = bg & (offset + rank_ex < _BUDGET)
    s_bg = jnp.sum(jnp.where(include, p, 0.0))
    n_inc = jnp.sum(include.astype(jnp.float32))

    acc_ref[0] += i_nb + jnp.float32(_BG) * s_bg
    acc_ref[1] += t_nb + s_bg + jnp.float32(_BG) * n_inc
    cnt_ref[0] += jnp.sum(bg.astype(jnp.int32))

    @pl.when(step == nsteps - 1)
    def _fin():
        inter = acc_ref[0]
        total = acc_ref[1]
        union = total - inter
        out_ref[0, 0] = 1.0 - (inter + 1.0) / (union + 1.0)


def kernel(inputs, targets):
    inp_r = inputs.reshape(4, _NCLS, _PLANE)
    sc_fn = pl.kernel(
        _argmax_sc,
        out_type=jax.ShapeDtypeStruct((_PX,), jnp.float32),
        mesh=plsc.VectorSubcoreMesh(
            core_axis_name="c", subcore_axis_name="s",
            num_cores=2, num_subcores=16),
        scratch_types=[
            pltpu.VMEM((_NCLS, _CHUNK), jnp.float32),
            pltpu.VMEM((_NCLS, _CHUNK), jnp.float32),
            pltpu.VMEM((_CHUNK,), jnp.float32),
            pltpu.VMEM((_CHUNK,), jnp.float32),
            pltpu.SemaphoreType.DMA,
            pltpu.SemaphoreType.DMA,
            pltpu.SemaphoreType.DMA,
            pltpu.SemaphoreType.DMA,
        ],
    )
    preds = sc_fn(inp_r)

    rows_total = _PX // 512
    p2 = preds.reshape(rows_total, 512)
    t2 = targets.reshape(rows_total, 512)
    out = pl.pallas_call(
        _gate_tc,
        grid=(rows_total // _ROWS,),
        in_specs=[
            pl.BlockSpec((_ROWS, 512), lambda i: (i, 0)),
            pl.BlockSpec((_ROWS, 512), lambda i: (i, 0)),
        ],
        out_specs=pl.BlockSpec(
            (1, 1), lambda i: (0, 0), memory_space=pltpu.SMEM),
        out_shape=jax.ShapeDtypeStruct((1, 1), jnp.float32),
        scratch_shapes=[
            pltpu.SMEM((2,), jnp.float32),
            pltpu.SMEM((1,), jnp.int32),
        ],
    )(p2, t2)
    return out[0, 0]


# speedup vs baseline: 1.0905x; 1.0905x over previous
import jax
import jax.numpy as jnp
from jax import lax
from jax.experimental import pallas as pl
from jax.experimental.pallas import tpu as pltpu
from jax.experimental.pallas import tpu_sc as plsc

_NCLS = 31
_BG = 30
_BUDGET = 80000.0
_PLANE = 512 * 512
_PX = 4 * _PLANE
_NW = 32
_PPW = _PX // _NW
_CHUNK = 1024
_NCH = _PPW // _CHUNK
_LANES = 16
_ROWS = 128


def _argmax_sc(inp_hbm, preds_hbm, buf0, buf1, pb0, pb1, si0, si1, so0, so1):
    cid = lax.axis_index("c")
    sid = lax.axis_index("s")
    wid = sid * 2 + cid
    base = wid * _PPW
    b = base // _PLANE
    off = base - b * _PLANE
    bufs = (buf0, buf1)
    pbs = (pb0, pb1)
    sis = (si0, si1)
    sos = (so0, so1)

    def in_copy(ci, k):
        return pltpu.make_async_copy(
            inp_hbm.at[b, :, pl.ds(off + ci * _CHUNK, _CHUNK)], bufs[k], sis[k])

    def out_copy(ci, k):
        return pltpu.make_async_copy(
            pbs[k], preds_hbm.at[pl.ds(base + ci * _CHUNK, _CHUNK)], sos[k])

    in_copy(0, 0).start()
    in_copy(1, 1).start()

    def pair(i2, carry):
        for k in range(2):
            ci = i2 * 2 + k
            in_copy(ci, k).wait()

            @pl.when(i2 >= 1)
            def _wait_out():
                out_copy(ci - 2, k).wait()

            buf = bufs[k]
            pb = pbs[k]

            @plsc.parallel_loop(0, _CHUNK // _LANES, unroll=4)
            def _grp(j):
                s = pl.ds(j * _LANES, _LANES)
                m = buf[0, s]
                idx = jnp.zeros((_LANES,), jnp.float32)
                for c in range(1, _NCLS):
                    v = buf[c, s]
                    gt = v > m
                    m = jnp.where(gt, v, m)
                    idx = jnp.where(gt, jnp.float32(c), idx)
                pb[s] = idx
            out_copy(ci, k).start()

            @pl.when(ci + 2 < _NCH)
            def _next_in():
                in_copy(ci + 2, k).start()

        return carry

    lax.fori_loop(0, _NCH // 2, pair, 0)
    out_copy(_NCH - 2, 0).wait()
    out_copy(_NCH - 1, 1).wait()


def _gate_tc(p_ref, t_ref, out_ref, acc_ref, cnt_ref):
    step = pl.program_id(0)
    nsteps = pl.num_programs(0)

    @pl.when(step == 0)
    def _init():
        acc_ref[0] = 0.0
        acc_ref[1] = 0.0
        cnt_ref[0] = 0

    p = p_ref[...]
    t = t_ref[...]
    tf = t.astype(jnp.float32)
    bg = t == _BG
    bgf = bg.astype(jnp.float32)

    i_nb = jnp.sum(jnp.where(bg, 0.0, p * tf))
    t_nb = jnp.sum(jnp.where(bg, 0.0, p + tf))

    rows, cols = bgf.shape
    jj = lax.broadcasted_iota(jnp.int32, (cols, cols), 0)
    kk = lax.broadcasted_iota(jnp.int32, (cols, cols), 1)
    tri_inc = (jj <= kk).astype(jnp.float32)
    cs_in = jnp.dot(bgf, tri_inc, preferred_element_type=jnp.float32)
    row_tot = cs_in[:, cols - 1:cols]
    ii = lax.broadcasted_iota(jnp.int32, (rows, rows), 0)
    ll = lax.broadcasted_iota(jnp.int32, (rows, rows), 1)
    tri_lo = (ll < ii).astype(jnp.float32)
    r_pref = jnp.dot(tri_lo, row_tot, preferred_element_type=jnp.float32)
    rank_ex = r_pref + (cs_in - bgf)

    offset = cnt_ref[0].astype(jnp.float32)
    include = bg & (offset + rank_ex < _BUDGET)
    s_bg = jnp.sum(jnp.where(include, p, 0.0))
    n_inc = jnp.sum(include.astype(jnp.float32))

    acc_ref[0] += i_nb + jnp.float32(_BG) * s_bg
    acc_ref[1] += t_nb + s_bg + jnp.float32(_BG) * n_inc
    cnt_ref[0] += jnp.sum(bg.astype(jnp.int32))

    @pl.when(step == nsteps - 1)
    def _fin():
        inter = acc_ref[0]
        total = acc_ref[1]
        union = total - inter
        out_ref[0, 0] = 1.0 - (inter + 1.0) / (union + 1.0)


def kernel(inputs, targets):
    inp_r = inputs.reshape(4, _NCLS, _PLANE)
    sc_fn = pl.kernel(
        _argmax_sc,
        out_type=jax.ShapeDtypeStruct((_PX,), jnp.float32),
        mesh=plsc.VectorSubcoreMesh(
            core_axis_name="c", subcore_axis_name="s",
            num_cores=2, num_subcores=16),
        scratch_types=[
            pltpu.VMEM((_NCLS, _CHUNK), jnp.float32),
            pltpu.VMEM((_NCLS, _CHUNK), jnp.float32),
            pltpu.VMEM((_CHUNK,), jnp.float32),
            pltpu.VMEM((_CHUNK,), jnp.float32),
            pltpu.SemaphoreType.DMA,
            pltpu.SemaphoreType.DMA,
            pltpu.SemaphoreType.DMA,
            pltpu.SemaphoreType.DMA,
        ],
    )
    preds = sc_fn(inp_r)

    rows_total = _PX // 512
    p2 = preds.reshape(rows_total, 512)
    t2 = targets.reshape(rows_total, 512)
    out = pl.pallas_call(
        _gate_tc,
        grid=(rows_total // _ROWS,),
        in_specs=[
            pl.BlockSpec((_ROWS, 512), lambda i: (i, 0)),
            pl.BlockSpec((_ROWS, 512), lambda i: (i, 0)),
        ],
        out_specs=pl.BlockSpec(
            (1, 1), lambda i: (0, 0), memory_space=pltpu.SMEM),
        out_shape=jax.ShapeDtypeStruct((1, 1), jnp.float32),
        scratch_shapes=[
            pltpu.SMEM((2,), jnp.float32),
            pltpu.SMEM((1,), jnp.int32),
        ],
    )(p2, t2)
    return out[0, 0]


# speedup vs baseline: 2.9186x; 2.6765x over previous
import jax
import jax.numpy as jnp
from jax import lax
from jax.experimental import pallas as pl
from jax.experimental.pallas import tpu as pltpu
from jax.experimental.pallas import tpu_sc as plsc

_NCLS = 31
_BG = 30
_BUDGET = 80000.0
_H = 512
_W = 512
_B = 4
_RTOT = _B * _H
_NW = 32
_RPW = _RTOT // _NW
_TR = 8
_TC = 128
_CGS = _W // _TC
_NCH = (_RPW // _TR) * _CGS
_LANES = 16
_ROWS = 128


def _argmax_sc(inp_hbm, preds_hbm, buf0, buf1, pb0, pb1, si0, si1, so0, so1):
    cid = lax.axis_index("c")
    sid = lax.axis_index("s")
    wid = sid * 2 + cid
    gr0 = wid * _RPW
    b = gr0 // _H
    r0 = gr0 - b * _H
    bufs = (buf0, buf1)
    pbs = (pb0, pb1)
    sis = (si0, si1)
    sos = (so0, so1)

    def in_copy(ci, k):
        rg = ci // _CGS
        cg = ci - rg * _CGS
        return pltpu.make_async_copy(
            inp_hbm.at[b, :, pl.ds(r0 + rg * _TR, _TR), pl.ds(cg * _TC, _TC)],
            bufs[k], sis[k])

    def out_copy(ci, k):
        rg = ci // _CGS
        cg = ci - rg * _CGS
        return pltpu.make_async_copy(
            pbs[k],
            preds_hbm.at[pl.ds(gr0 + rg * _TR, _TR), pl.ds(cg * _TC, _TC)],
            sos[k])

    in_copy(0, 0).start()
    in_copy(1, 1).start()

    def pair(i2, carry):
        for k in range(2):
            ci = i2 * 2 + k
            in_copy(ci, k).wait()

            @pl.when(i2 >= 1)
            def _wait_out():
                out_copy(ci - 2, k).wait()

            buf = bufs[k]
            pb = pbs[k]

            @plsc.parallel_loop(0, (_TR * _TC) // _LANES, unroll=4)
            def _grp(j):
                r = j // (_TC // _LANES)
                col = (j - r * (_TC // _LANES)) * _LANES
                s = pl.ds(col, _LANES)
                m = buf[0, r, s]
                idx = jnp.zeros((_LANES,), jnp.float32)
                for c in range(1, _NCLS):
                    v = buf[c, r, s]
                    gt = v > m
                    m = jnp.where(gt, v, m)
                    idx = jnp.where(gt, jnp.float32(c), idx)
                pb[r, s] = idx

            out_copy(ci, k).start()

            @pl.when(ci + 2 < _NCH)
            def _next_in():
                in_copy(ci + 2, k).start()

        return carry

    lax.fori_loop(0, _NCH // 2, pair, 0)
    out_copy(_NCH - 2, 0).wait()
    out_copy(_NCH - 1, 1).wait()


def _gate_tc(p_ref, t_ref, out_ref, acc_ref, cnt_ref):
    step = pl.program_id(0)
    nsteps = pl.num_programs(0)

    @pl.when(step == 0)
    def _init():
        acc_ref[0] = 0.0
        acc_ref[1] = 0.0
        cnt_ref[0] = 0

    p = p_ref[...]
    t = t_ref[...]
    tf = t.astype(jnp.float32)
    bg = t == _BG
    bgf = bg.astype(jnp.float32)

    i_nb = jnp.sum(jnp.where(bg, 0.0, p * tf))
    t_nb = jnp.sum(jnp.where(bg, 0.0, p + tf))

    rows, cols = bgf.shape
    jj = lax.broadcasted_iota(jnp.int32, (cols, cols), 0)
    kk = lax.broadcasted_iota(jnp.int32, (cols, cols), 1)
    tri_inc = (jj <= kk).astype(jnp.float32)
    cs_in = jnp.dot(bgf, tri_inc, preferred_element_type=jnp.float32)
    row_tot = cs_in[:, cols - 1:cols]
    ii = lax.broadcasted_iota(jnp.int32, (rows, rows), 0)
    ll = lax.broadcasted_iota(jnp.int32, (rows, rows), 1)
    tri_lo = (ll < ii).astype(jnp.float32)
    r_pref = jnp.dot(tri_lo, row_tot, preferred_element_type=jnp.float32)
    rank_ex = r_pref + (cs_in - bgf)

    offset = cnt_ref[0].astype(jnp.float32)
    include = bg & (offset + rank_ex < _BUDGET)
    s_bg = jnp.sum(jnp.where(include, p, 0.0))
    n_inc = jnp.sum(include.astype(jnp.float32))

    acc_ref[0] += i_nb + jnp.float32(_BG) * s_bg
    acc_ref[1] += t_nb + s_bg + jnp.float32(_BG) * n_inc
    cnt_ref[0] += jnp.sum(bg.astype(jnp.int32))

    @pl.when(step == nsteps - 1)
    def _fin():
        inter = acc_ref[0]
        total = acc_ref[1]
        union = total - inter
        out_ref[0, 0] = 1.0 - (inter + 1.0) / (union + 1.0)


def kernel(inputs, targets):
    sc_fn = pl.kernel(
        _argmax_sc,
        out_type=jax.ShapeDtypeStruct((_RTOT, _W), jnp.float32),
        mesh=plsc.VectorSubcoreMesh(
            core_axis_name="c", subcore_axis_name="s",
            num_cores=2, num_subcores=16),
        scratch_types=[
            pltpu.VMEM((_NCLS, _TR, _TC), jnp.float32),
            pltpu.VMEM((_NCLS, _TR, _TC), jnp.float32),
            pltpu.VMEM((_TR, _TC), jnp.float32),
            pltpu.VMEM((_TR, _TC), jnp.float32),
            pltpu.SemaphoreType.DMA,
            pltpu.SemaphoreType.DMA,
            pltpu.SemaphoreType.DMA,
            pltpu.SemaphoreType.DMA,
        ],
    )
    preds = sc_fn(inputs)

    t2 = targets.reshape(_RTOT, _W)
    out = pl.pallas_call(
        _gate_tc,
        grid=(_RTOT // _ROWS,),
        in_specs=[
            pl.BlockSpec((_ROWS, _W), lambda i: (i, 0)),
            pl.BlockSpec((_ROWS, _W), lambda i: (i, 0)),
        ],
        out_specs=pl.BlockSpec(
            (1, 1), lambda i: (0, 0), memory_space=pltpu.SMEM),
        out_shape=jax.ShapeDtypeStruct((1, 1), jnp.float32),
        scratch_shapes=[
            pltpu.SMEM((2,), jnp.float32),
            pltpu.SMEM((1,), jnp.int32),
        ],
    )(preds, t2)
    return out[0, 0]


# speedup vs baseline: 5.5857x; 1.9139x over previous
import jax
import jax.numpy as jnp
from jax.experimental import pallas as pl
from jax.experimental.pallas import tpu as pltpu

_NCLS = 31
_BG = 30
_BUDGET = 80000.0
_ROWS = 128


def _iou_kernel(x_ref, t_ref, out_ref, acc_ref, cnt_ref):
    b = pl.program_id(0)
    r = pl.program_id(1)
    nb = pl.num_programs(1)
    step = b * nb + r
    nsteps = pl.num_programs(0) * nb

    @pl.when(step == 0)
    def _init():
        acc_ref[0] = 0.0
        acc_ref[1] = 0.0
        cnt_ref[0] = 0

    x = x_ref[0]
    m = x[0]
    idx = jnp.zeros_like(m)
    for c in range(1, _NCLS):
        xc = x[c]
        gt = xc > m
        m = jnp.where(gt, xc, m)
        idx = jnp.where(gt, jnp.float32(c), idx)
    p = idx

    t = t_ref[0]
    tf = t.astype(jnp.float32)
    bg = t == _BG
    bgf = bg.astype(jnp.float32)

    i_nb = jnp.sum(jnp.where(bg, 0.0, p * tf))
    t_nb = jnp.sum(jnp.where(bg, 0.0, p + tf))

    cntf = jnp.sum(bgf)
    s_all = jnp.sum(jnp.where(bg, p, 0.0))
    offset = cnt_ref[0].astype(jnp.float32)
    all_in = offset + cntf <= _BUDGET

    @pl.when(all_in)
    def _fast():
        acc_ref[0] += i_nb + jnp.float32(_BG) * s_all
        acc_ref[1] += t_nb + s_all + jnp.float32(_BG) * cntf

    @pl.when(jnp.logical_not(all_in))
    def _ranked():
        rows, cols = bgf.shape
        jj = jax.lax.broadcasted_iota(jnp.int32, (cols, cols), 0)
        kk = jax.lax.broadcasted_iota(jnp.int32, (cols, cols), 1)
        tri_inc = (jj <= kk).astype(jnp.float32)
        cs_in = jnp.dot(bgf, tri_inc, preferred_element_type=jnp.float32)
        row_tot = cs_in[:, cols - 1:cols]
        ii = jax.lax.broadcasted_iota(jnp.int32, (rows, rows), 0)
        ll = jax.lax.broadcasted_iota(jnp.int32, (rows, rows), 1)
        tri_lo = (ll < ii).astype(jnp.float32)
        r_pref = jnp.dot(tri_lo, row_tot, preferred_element_type=jnp.float32)
        rank_ex = r_pref + (cs_in - bgf)

        include = bg & (offset + rank_ex < _BUDGET)
        s_bg = jnp.sum(jnp.where(include, p, 0.0))
        n_inc = jnp.sum(include.astype(jnp.float32))
        acc_ref[0] += i_nb + jnp.float32(_BG) * s_bg
        acc_ref[1] += t_nb + s_bg + jnp.float32(_BG) * n_inc

    cnt_ref[0] += jnp.sum(bg.astype(jnp.int32))

    @pl.when(step == nsteps - 1)
    def _fin():
        inter = acc_ref[0]
        total = acc_ref[1]
        union = total - inter
        out_ref[0, 0] = 1.0 - (inter + 1.0) / (union + 1.0)


def kernel(inputs, targets):
    b, ncls, h, w = inputs.shape
    nb = h // _ROWS
    out = pl.pallas_call(
        _iou_kernel,
        grid=(b, nb),
        in_specs=[
            pl.BlockSpec((1, ncls, _ROWS, w), lambda i, j: (i, 0, j, 0)),
            pl.BlockSpec((1, _ROWS, w), lambda i, j: (i, j, 0)),
        ],
        out_specs=pl.BlockSpec(
            (1, 1), lambda i, j: (0, 0), memory_space=pltpu.SMEM),
        out_shape=jax.ShapeDtypeStruct((1, 1), jnp.float32),
        scratch_shapes=[
            pltpu.SMEM((2,), jnp.float32),
            pltpu.SMEM((1,), jnp.int32),
        ],
    )(inputs, targets)
    return out[0, 0]
